# Initial kernel scaffold; baseline (speedup 1.0000x reference)
import jax
import jax.numpy as jnp
from jax.experimental import pallas as pl


def kernel(x, edge_index, W1, b1, W2, b2):
    raise NotImplementedError("write your pallas kernel here")


# speedup vs baseline: 15.7299x; 15.7299x over previous
import functools

import jax
import jax.numpy as jnp
from jax import lax
from jax.experimental import pallas as pl
from jax.experimental.pallas import tpu as pltpu
from jax.experimental.pallas import tpu_sc as plsc

NC = 2
NS = 16
NW = NC * NS
CHUNK = 128
D = 128


def _sc_meshes():
    mesh = plsc.VectorSubcoreMesh(core_axis_name="c", subcore_axis_name="s")
    return mesh


def _make_prop(np_, nchunk):
    rpt = np_ // NS
    mesh = _sc_meshes()

    @functools.partial(
        pl.kernel,
        out_type=jax.ShapeDtypeStruct((NC * np_, D), jnp.float32),
        mesh=mesh,
        scratch_types=[
            pltpu.VMEM_SHARED((np_, D), jnp.float32),
            pltpu.VMEM((nchunk, CHUNK), jnp.int32),
            pltpu.VMEM((nchunk, CHUNK), jnp.int32),
            pltpu.VMEM((CHUNK, D), jnp.float32),
            pltpu.SemaphoreType.DMA,
        ],
    )
    def prop(g_hbm, src_hbm, dst_hbm, zrows_hbm, out_hbm,
             acc, src_v, dst_v, rows_v, sem):
        c = lax.axis_index("c")
        s = lax.axis_index("s")
        wid = c * NS + s
        pltpu.sync_copy(zrows_hbm, acc.at[pl.ds(s * rpt, rpt)])
        pltpu.sync_copy(src_hbm.at[wid], src_v)
        pltpu.sync_copy(dst_hbm.at[wid], dst_v)
        plsc.subcore_barrier()

        def chunk(j, carry):
            pltpu.async_copy(g_hbm.at[src_v.at[j]], rows_v, sem).wait()
            pltpu.sync_copy(rows_v, acc.at[dst_v.at[j]], add=True)
            return carry

        lax.fori_loop(0, nchunk, chunk, 0)
        plsc.subcore_barrier()
        pltpu.sync_copy(acc.at[pl.ds(s * rpt, rpt)],
                        out_hbm.at[pl.ds(c * np_ + s * rpt, rpt)])

    return prop


def _make_deg(np_, nchunk):
    rpt = np_ // NS
    mesh = _sc_meshes()

    @functools.partial(
        pl.kernel,
        out_type=jax.ShapeDtypeStruct((NC * np_, D), jnp.float32),
        mesh=mesh,
        scratch_types=[
            pltpu.VMEM_SHARED((np_, D), jnp.float32),
            pltpu.VMEM((nchunk, CHUNK), jnp.int32),
            pltpu.VMEM((CHUNK, D), jnp.float32),
        ],
    )
    def deg(dst_hbm, ones_hbm, zrows_hbm, out_hbm, acc, dst_v, ones_v):
        c = lax.axis_index("c")
        s = lax.axis_index("s")
        wid = c * NS + s
        pltpu.sync_copy(zrows_hbm, acc.at[pl.ds(s * rpt, rpt)])
        pltpu.sync_copy(dst_hbm.at[wid], dst_v)
        pltpu.sync_copy(ones_hbm, ones_v)
        plsc.subcore_barrier()

        def chunk(j, carry):
            pltpu.sync_copy(ones_v, acc.at[dst_v.at[j]], add=True)
            return carry

        lax.fori_loop(0, nchunk, chunk, 0)
        plsc.subcore_barrier()
        pltpu.sync_copy(acc.at[pl.ds(s * rpt, rpt)],
                        out_hbm.at[pl.ds(c * np_ + s * rpt, rpt)])

    return deg


def _mm_scale_body(x_ref, w_ref, dm_ref, o_ref):
    o_ref[...] = dm_ref[...] * jnp.dot(x_ref[...], w_ref[...],
                                       preferred_element_type=jnp.float32,
                                       precision=lax.Precision.HIGHEST)


def _mid_body(p0_ref, p1_ref, g_ref, dm_ref, b_ref, w_ref, o_ref):
    t = dm_ref[...] * (p0_ref[...] + p1_ref[...] + g_ref[...]) + b_ref[...]
    t = jnp.maximum(t, 0.0)
    o_ref[...] = dm_ref[...] * jnp.dot(t, w_ref[...],
                                       preferred_element_type=jnp.float32,
                                       precision=lax.Precision.HIGHEST)


def _fin_body(p0_ref, p1_ref, g_ref, dm_ref, b_ref, o_ref):
    o_ref[...] = (dm_ref[...] * (p0_ref[...] + p1_ref[...] + g_ref[...])
                  + b_ref[...])


def _row_spec(r):
    return pl.BlockSpec((r, D), lambda i: (i, 0))


def _full_spec(shape):
    return pl.BlockSpec(shape, lambda i: tuple(0 for _ in shape))


def kernel(x, edge_index, W1, b1, W2, b2):
    n, d_in = x.shape
    assert d_in == D
    e = edge_index.shape[1]

    np_ = ((n + NS * 8 - 1) // (NS * 8)) * (NS * 8)
    if np_ == n:
        np_ += NS * 8
    e_pad = ((e + NW * CHUNK - 1) // (NW * CHUNK)) * (NW * CHUNK)
    nchunk = e_pad // (NW * CHUNK)
    rblk = 1024 if np_ % 1024 == 0 else NS * 8

    src = edge_index[0].astype(jnp.int32)
    dst = edge_index[1].astype(jnp.int32)
    pad = n + (jnp.arange(e_pad - e, dtype=jnp.int32) % (np_ - n))
    src3 = jnp.concatenate([src, pad]).reshape(NW, nchunk, CHUNK)
    dst3 = jnp.concatenate([dst, pad]).reshape(NW, nchunk, CHUNK)
    x_pad = jnp.pad(x, ((0, np_ - n), (0, 0)))

    rpt = np_ // NS
    zrows = jnp.zeros((rpt, D), jnp.float32)
    ones = jnp.ones((CHUNK, D), jnp.float32)

    deg_k = _make_deg(np_, nchunk)
    prop_k = _make_prop(np_, nchunk)

    degp = deg_k(dst3, ones, zrows)
    deg = degp[:np_, 0] + degp[np_:, 0] + 1.0
    dmat = jnp.broadcast_to(lax.rsqrt(deg)[:, None], (np_, D))

    grid = (np_ // rblk,)
    g1 = pl.pallas_call(
        _mm_scale_body,
        grid=grid,
        in_specs=[_row_spec(rblk), _full_spec((D, D)), _row_spec(rblk)],
        out_specs=_row_spec(rblk),
        out_shape=jax.ShapeDtypeStruct((np_, D), jnp.float32),
    )(x_pad, W1, dmat)

    parts1 = prop_k(g1, src3, dst3, zrows)

    g2 = pl.pallas_call(
        _mid_body,
        grid=grid,
        in_specs=[_row_spec(rblk), _row_spec(rblk), _row_spec(rblk),
                  _row_spec(rblk), _full_spec((1, D)), _full_spec((D, D))],
        out_specs=_row_spec(rblk),
        out_shape=jax.ShapeDtypeStruct((np_, D), jnp.float32),
    )(parts1[:np_], parts1[np_:], g1, dmat, b1.reshape(1, D), W2)

    parts2 = prop_k(g2, src3, dst3, zrows)

    out = pl.pallas_call(
        _fin_body,
        grid=grid,
        in_specs=[_row_spec(rblk), _row_spec(rblk), _row_spec(rblk),
                  _row_spec(rblk), _full_spec((1, D))],
        out_specs=_row_spec(rblk),
        out_shape=jax.ShapeDtypeStruct((np_, D), jnp.float32),
    )(parts2[:np_], parts2[np_:], g2, dmat, b2.reshape(1, D))

    return out[:n]


# speedup vs baseline: 19.1582x; 1.2180x over previous
import functools

import jax
import jax.numpy as jnp
from jax import lax
from jax.experimental import pallas as pl
from jax.experimental.pallas import tpu as pltpu
from jax.experimental.pallas import tpu_sc as plsc

NC = 2
NS = 16
NW = NC * NS
CHUNK = 128
IBLK = 8
D = 128


def _make_prop(np_, nchunk):
    rpt = np_ // NS
    nblk = nchunk // IBLK
    mesh = plsc.VectorSubcoreMesh(core_axis_name="c", subcore_axis_name="s")

    @functools.partial(
        pl.kernel,
        out_type=jax.ShapeDtypeStruct((NC * np_, D), jnp.float32),
        mesh=mesh,
        scratch_types=[
            pltpu.VMEM_SHARED((np_, D), jnp.float32),
            pltpu.VMEM((2, IBLK, 2, CHUNK), jnp.int32),
            pltpu.VMEM((CHUNK, D), jnp.float32),
            pltpu.VMEM((CHUNK, D), jnp.float32),
            pltpu.SemaphoreType.DMA,
            pltpu.SemaphoreType.DMA,
            pltpu.SemaphoreType.DMA,
            pltpu.SemaphoreType.DMA,
        ],
    )
    def prop(g_hbm, e_hbm, zrows_hbm, out_hbm,
             acc, idx_v, buf0, buf1, is0, is1, gs0, gs1):
        bufs, gsems, isems = (buf0, buf1), (gs0, gs1), (is0, is1)
        c = lax.axis_index("c")
        s = lax.axis_index("s")
        wid = c * NS + s
        pltpu.sync_copy(zrows_hbm, acc.at[pl.ds(s * rpt, rpt)])

        def idx_copy(bk, slot):
            return pltpu.make_async_copy(
                e_hbm.at[wid].at[pl.ds(bk * IBLK, IBLK)],
                idx_v.at[slot], isems[slot])

        def gather(bk, k, b):
            slot = bk % 2
            return pltpu.make_async_copy(
                g_hbm.at[idx_v.at[slot, k, 0]], bufs[b], gsems[b])

        idx_copy(0, 0).start()
        idx_copy(1, 1).start()
        plsc.subcore_barrier()

        def do_block(bk, slot):
            idx_copy(bk, slot).wait()
            gather(bk, 0, 0).start()
            for k in range(IBLK):
                b = k % 2
                if k + 1 < IBLK:
                    gather(bk, k + 1, (k + 1) % 2).start()
                gather(bk, k, b).wait()
                pltpu.sync_copy(bufs[b], acc.at[idx_v.at[slot, k, 1]],
                                add=True)
            pl.when(bk + 2 < nblk)(lambda: idx_copy(bk + 2, slot).start())

        def outer(i, carry):
            do_block(2 * i, 0)
            do_block(2 * i + 1, 1)
            return carry

        lax.fori_loop(0, nblk // 2, outer, 0)
        plsc.subcore_barrier()
        pltpu.sync_copy(acc.at[pl.ds(s * rpt, rpt)],
                        out_hbm.at[pl.ds(c * np_ + s * rpt, rpt)])

    return prop


def _make_deg(np_, nchunk):
    rpt = np_ // NS
    ndeep = 4
    mesh = plsc.VectorSubcoreMesh(core_axis_name="c", subcore_axis_name="s")

    @functools.partial(
        pl.kernel,
        out_type=jax.ShapeDtypeStruct((NC * np_, D), jnp.float32),
        mesh=mesh,
        scratch_types=[
            pltpu.VMEM_SHARED((np_, D), jnp.float32),
            pltpu.VMEM((nchunk, 2, CHUNK), jnp.int32),
            pltpu.VMEM((CHUNK, D), jnp.float32),
        ] + [pltpu.SemaphoreType.DMA] * 4,
    )
    def deg(e_hbm, ones_hbm, zrows_hbm, out_hbm, acc, idx_v, ones_v, *sems):
        c = lax.axis_index("c")
        s = lax.axis_index("s")
        wid = c * NS + s
        pltpu.sync_copy(zrows_hbm, acc.at[pl.ds(s * rpt, rpt)])
        pltpu.sync_copy(e_hbm.at[wid], idx_v)
        pltpu.sync_copy(ones_hbm, ones_v)
        plsc.subcore_barrier()

        def scat(j, b):
            return pltpu.make_async_copy(ones_v, acc.at[idx_v.at[j, 1]],
                                         sems[b])

        for b in range(ndeep):
            scat(b, b).start(add=True)

        def outer(i, carry):
            for b in range(ndeep):
                j = i * ndeep + b
                scat(j, b).wait()
                nj = j + ndeep
                pl.when(nj < nchunk)(
                    lambda nj=nj, b=b: scat(nj, b).start(add=True))
            return carry

        lax.fori_loop(0, nchunk // ndeep, outer, 0)
        plsc.subcore_barrier()
        pltpu.sync_copy(acc.at[pl.ds(s * rpt, rpt)],
                        out_hbm.at[pl.ds(c * np_ + s * rpt, rpt)])

    return deg


def _mm_scale_body(x_ref, w_ref, dp0_ref, dp1_ref, o_ref, dm_ref):
    dm = lax.rsqrt(dp0_ref[...] + dp1_ref[...] + 1.0)
    dm_ref[...] = dm
    o_ref[...] = dm * jnp.dot(x_ref[...], w_ref[...],
                              preferred_element_type=jnp.float32,
                              precision=lax.Precision.HIGHEST)


def _mid_body(p0_ref, p1_ref, g_ref, dm_ref, b_ref, w_ref, o_ref):
    t = dm_ref[...] * (p0_ref[...] + p1_ref[...] + g_ref[...]) + b_ref[...]
    t = jnp.maximum(t, 0.0)
    o_ref[...] = dm_ref[...] * jnp.dot(t, w_ref[...],
                                       preferred_element_type=jnp.float32,
                                       precision=lax.Precision.HIGHEST)


def _fin_body(p0_ref, p1_ref, g_ref, dm_ref, b_ref, o_ref):
    o_ref[...] = (dm_ref[...] * (p0_ref[...] + p1_ref[...] + g_ref[...])
                  + b_ref[...])


def _row_spec(r):
    return pl.BlockSpec((r, D), lambda i: (i, 0))


def _full_spec(shape):
    return pl.BlockSpec(shape, lambda i: tuple(0 for _ in shape))


def kernel(x, edge_index, W1, b1, W2, b2):
    n, d_in = x.shape
    assert d_in == D
    e = edge_index.shape[1]

    np_ = ((n + NS * 8 - 1) // (NS * 8)) * (NS * 8)
    if np_ == n:
        np_ += NS * 8
    estep = NW * CHUNK * IBLK * 2
    e_pad = ((e + estep - 1) // estep) * estep
    nchunk = e_pad // (NW * CHUNK)
    rblk = 1024 if np_ % 1024 == 0 else NS * 8

    src = edge_index[0].astype(jnp.int32)
    dst = edge_index[1].astype(jnp.int32)
    pad = n + (jnp.arange(e_pad - e, dtype=jnp.int32) % (np_ - n))
    src3 = jnp.concatenate([src, pad]).reshape(NW, nchunk, 1, CHUNK)
    dst3 = jnp.concatenate([dst, pad]).reshape(NW, nchunk, 1, CHUNK)
    e3 = jnp.concatenate([src3, dst3], axis=2)
    x_pad = jnp.pad(x, ((0, np_ - n), (0, 0)))

    rpt = np_ // NS
    zrows = jnp.zeros((rpt, D), jnp.float32)
    ones = jnp.ones((CHUNK, D), jnp.float32)

    deg_k = _make_deg(np_, nchunk)
    prop_k = _make_prop(np_, nchunk)

    degp = deg_k(e3, ones, zrows)

    grid = (np_ // rblk,)
    g1, dmat = pl.pallas_call(
        _mm_scale_body,
        grid=grid,
        in_specs=[_row_spec(rblk), _full_spec((D, D)),
                  _row_spec(rblk), _row_spec(rblk)],
        out_specs=[_row_spec(rblk), _row_spec(rblk)],
        out_shape=[jax.ShapeDtypeStruct((np_, D), jnp.float32),
                   jax.ShapeDtypeStruct((np_, D), jnp.float32)],
    )(x_pad, W1, degp[:np_], degp[np_:])

    parts1 = prop_k(g1, e3, zrows)

    g2 = pl.pallas_call(
        _mid_body,
        grid=grid,
        in_specs=[_row_spec(rblk), _row_spec(rblk), _row_spec(rblk),
                  _row_spec(rblk), _full_spec((1, D)), _full_spec((D, D))],
        out_specs=_row_spec(rblk),
        out_shape=jax.ShapeDtypeStruct((np_, D), jnp.float32),
    )(parts1[:np_], parts1[np_:], g1, dmat, b1.reshape(1, D), W2)

    parts2 = prop_k(g2, e3, zrows)

    out = pl.pallas_call(
        _fin_body,
        grid=grid,
        in_specs=[_row_spec(rblk), _row_spec(rblk), _row_spec(rblk),
                  _row_spec(rblk), _full_spec((1, D))],
        out_specs=_row_spec(rblk),
        out_shape=jax.ShapeDtypeStruct((np_, D), jnp.float32),
    )(parts2[:np_], parts2[np_:], g2, dmat, b2.reshape(1, D))

    return out[:n]


# speedup vs baseline: 25.7715x; 1.3452x over previous
import functools

import jax
import jax.numpy as jnp
from jax import lax
from jax.experimental import pallas as pl
from jax.experimental.pallas import tpu as pltpu
from jax.experimental.pallas import tpu_sc as plsc

NC = 2
NS = 16
NW = NC * NS
CHUNK = 128
IBLK = 8
D = 128


def _make_prop(np_, nchunk):
    rpt = np_ // NS
    nblk = nchunk // IBLK
    mesh = plsc.VectorSubcoreMesh(core_axis_name="c", subcore_axis_name="s")

    @functools.partial(
        pl.kernel,
        out_type=jax.ShapeDtypeStruct((NC, np_, D), jnp.float32),
        mesh=mesh,
        scratch_types=[
            pltpu.VMEM_SHARED((np_, D), jnp.float32),
            pltpu.VMEM((2, IBLK, 2, CHUNK), jnp.int32),
            pltpu.VMEM((CHUNK, D), jnp.float32),
            pltpu.VMEM((CHUNK, D), jnp.float32),
            pltpu.SemaphoreType.DMA,
            pltpu.SemaphoreType.DMA,
            pltpu.SemaphoreType.DMA,
            pltpu.SemaphoreType.DMA,
        ],
    )
    def prop(g_hbm, e_hbm, zrows_hbm, out_hbm,
             acc, idx_v, buf0, buf1, is0, is1, gs0, gs1):
        bufs, gsems, isems = (buf0, buf1), (gs0, gs1), (is0, is1)
        c = lax.axis_index("c")
        s = lax.axis_index("s")
        wid = c * NS + s
        pltpu.sync_copy(zrows_hbm, acc.at[pl.ds(s * rpt, rpt)])

        def idx_copy(bk, slot):
            return pltpu.make_async_copy(
                e_hbm.at[wid].at[pl.ds(bk * IBLK, IBLK)],
                idx_v.at[slot], isems[slot])

        def gather(bk, k, b):
            slot = bk % 2
            return pltpu.make_async_copy(
                g_hbm.at[idx_v.at[slot, k, 0]], bufs[b], gsems[b])

        idx_copy(0, 0).start()
        idx_copy(1, 1).start()
        plsc.subcore_barrier()

        def do_block(bk, slot):
            idx_copy(bk, slot).wait()
            gather(bk, 0, 0).start()
            for k in range(IBLK):
                b = k % 2
                if k + 1 < IBLK:
                    gather(bk, k + 1, (k + 1) % 2).start()
                gather(bk, k, b).wait()
                pltpu.sync_copy(bufs[b], acc.at[idx_v.at[slot, k, 1]],
                                add=True)
            pl.when(bk + 2 < nblk)(lambda: idx_copy(bk + 2, slot).start())

        def outer(i, carry):
            do_block(2 * i, 0)
            do_block(2 * i + 1, 1)
            return carry

        lax.fori_loop(0, nblk // 2, outer, 0)
        plsc.subcore_barrier()
        pltpu.sync_copy(acc.at[pl.ds(s * rpt, rpt)],
                        out_hbm.at[c, pl.ds(s * rpt, rpt)])

    return prop


def _make_deg(np_, nchunk):
    rpt = np_ // NS
    ndeep = 4
    mesh = plsc.VectorSubcoreMesh(core_axis_name="c", subcore_axis_name="s")

    @functools.partial(
        pl.kernel,
        out_type=jax.ShapeDtypeStruct((NC, np_, D), jnp.float32),
        mesh=mesh,
        scratch_types=[
            pltpu.VMEM_SHARED((np_, D), jnp.float32),
            pltpu.VMEM((nchunk, 2, CHUNK), jnp.int32),
            pltpu.VMEM((CHUNK, D), jnp.float32),
        ] + [pltpu.SemaphoreType.DMA] * 4,
    )
    def deg(e_hbm, ones_hbm, zrows_hbm, out_hbm, acc, idx_v, ones_v, *sems):
        c = lax.axis_index("c")
        s = lax.axis_index("s")
        wid = c * NS + s
        pltpu.sync_copy(zrows_hbm, acc.at[pl.ds(s * rpt, rpt)])
        pltpu.sync_copy(e_hbm.at[wid], idx_v)
        pltpu.sync_copy(ones_hbm, ones_v)
        plsc.subcore_barrier()

        def scat(j, b):
            return pltpu.make_async_copy(ones_v, acc.at[idx_v.at[j, 1]],
                                         sems[b])

        for b in range(ndeep):
            scat(b, b).start(add=True)

        def outer(i, carry):
            for b in range(ndeep):
                j = i * ndeep + b
                scat(j, b).wait()
                nj = j + ndeep
                pl.when(nj < nchunk)(
                    lambda nj=nj, b=b: scat(nj, b).start(add=True))
            return carry

        lax.fori_loop(0, nchunk // ndeep, outer, 0)
        plsc.subcore_barrier()
        pltpu.sync_copy(acc.at[pl.ds(s * rpt, rpt)],
                        out_hbm.at[c, pl.ds(s * rpt, rpt)])

    return deg


def _mm_scale_body(x_ref, w_ref, d_ref, o_ref):
    o_ref[...] = d_ref[...] * jnp.dot(x_ref[...], w_ref[...],
                                      preferred_element_type=jnp.float32,
                                      precision=lax.Precision.HIGHEST)


def _mid_body(p_ref, g_ref, d_ref, b_ref, w_ref, o_ref):
    t = d_ref[...] * (p_ref[0] + p_ref[1] + g_ref[...]) + b_ref[...]
    t = jnp.maximum(t, 0.0)
    o_ref[...] = d_ref[...] * jnp.dot(t, w_ref[...],
                                      preferred_element_type=jnp.float32,
                                      precision=lax.Precision.HIGHEST)


def _fin_body(p_ref, g_ref, d_ref, b_ref, o_ref):
    o_ref[...] = (d_ref[...] * (p_ref[0] + p_ref[1] + g_ref[...])
                  + b_ref[...])


def _row_spec(r):
    return pl.BlockSpec((r, D), lambda i: (i, 0))


def _col_spec(r):
    return pl.BlockSpec((r, 1), lambda i: (i, 0))


def _part_spec(r):
    return pl.BlockSpec((NC, r, D), lambda i: (0, i, 0))


def _full_spec(shape):
    return pl.BlockSpec(shape, lambda i: tuple(0 for _ in shape))


def kernel(x, edge_index, W1, b1, W2, b2):
    n, d_in = x.shape
    assert d_in == D
    e = edge_index.shape[1]

    np_ = ((n + NS * 8 - 1) // (NS * 8)) * (NS * 8)
    if np_ == n:
        np_ += NS * 8
    estep = NW * CHUNK * IBLK * 2
    e_pad = ((e + estep - 1) // estep) * estep
    nchunk = e_pad // (NW * CHUNK)
    rb = n // 5 if n % 5 == 0 and (n // 5) % 8 == 0 else None
    if rb is None:
        rb = 8
        while n % (rb * 2) == 0 and rb < 2048:
            rb *= 2

    src = edge_index[0].astype(jnp.int32)
    dst = edge_index[1].astype(jnp.int32)
    pad = n + (jnp.arange(e_pad - e, dtype=jnp.int32) % (np_ - n))
    src3 = jnp.concatenate([src, pad]).reshape(NW, nchunk, 1, CHUNK)
    dst3 = jnp.concatenate([dst, pad]).reshape(NW, nchunk, 1, CHUNK)
    e3 = jnp.concatenate([src3, dst3], axis=2)

    rpt = np_ // NS
    zrows = jnp.zeros((rpt, D), jnp.float32)
    ones = jnp.ones((CHUNK, D), jnp.float32)

    deg_k = _make_deg(np_, nchunk)
    prop_k = _make_prop(np_, nchunk)

    degp = deg_k(e3, ones, zrows)
    dcol = lax.rsqrt(degp[0, :, 0] + degp[1, :, 0] + 1.0)[:, None]

    grid = (n // rb,)
    g1 = pl.pallas_call(
        _mm_scale_body,
        grid=grid,
        in_specs=[_row_spec(rb), _full_spec((D, D)), _col_spec(rb)],
        out_specs=_row_spec(rb),
        out_shape=jax.ShapeDtypeStruct((np_, D), jnp.float32),
    )(x, W1, dcol)

    parts1 = prop_k(g1, e3, zrows)

    g2 = pl.pallas_call(
        _mid_body,
        grid=grid,
        in_specs=[_part_spec(rb), _row_spec(rb), _col_spec(rb),
                  _full_spec((1, D)), _full_spec((D, D))],
        out_specs=_row_spec(rb),
        out_shape=jax.ShapeDtypeStruct((np_, D), jnp.float32),
    )(parts1, g1, dcol, b1.reshape(1, D), W2)

    parts2 = prop_k(g2, e3, zrows)

    out = pl.pallas_call(
        _fin_body,
        grid=grid,
        in_specs=[_part_spec(rb), _row_spec(rb), _col_spec(rb),
                  _full_spec((1, D))],
        out_specs=_row_spec(rb),
        out_shape=jax.ShapeDtypeStruct((n, D), jnp.float32),
    )(parts2, g2, dcol, b2.reshape(1, D))

    return out


# speedup vs baseline: 26.5746x; 1.0312x over previous
import functools

import jax
import jax.numpy as jnp
from jax import lax
from jax.experimental import pallas as pl
from jax.experimental.pallas import tpu as pltpu
from jax.experimental.pallas import tpu_sc as plsc

NC = 2
NS = 16
NW = NC * NS
CHUNK = 128
IBLK = 8
D = 128


def _make_prop(np_, nchunk):
    rpt = np_ // NS
    nblk = nchunk // IBLK
    mesh = plsc.VectorSubcoreMesh(core_axis_name="c", subcore_axis_name="s")

    @functools.partial(
        pl.kernel,
        out_type=jax.ShapeDtypeStruct((NC, np_, D), jnp.float32),
        mesh=mesh,
        scratch_types=[
            pltpu.VMEM_SHARED((np_, D), jnp.float32),
            pltpu.VMEM((2, 2, IBLK, CHUNK), jnp.int32),
            pltpu.VMEM((CHUNK, D), jnp.float32),
            pltpu.VMEM((CHUNK, D), jnp.float32),
            pltpu.SemaphoreType.DMA,
            pltpu.SemaphoreType.DMA,
            pltpu.SemaphoreType.DMA,
            pltpu.SemaphoreType.DMA,
        ],
    )
    def prop(g_hbm, e_hbm, zrows_hbm, out_hbm,
             acc, idx_v, buf0, buf1, is0, is1, gs0, gs1):
        bufs, gsems, isems = (buf0, buf1), (gs0, gs1), (is0, is1)
        c = lax.axis_index("c")
        s = lax.axis_index("s")
        wid = c * NS + s
        pltpu.sync_copy(zrows_hbm, acc.at[pl.ds(s * rpt, rpt)])

        def idx_copies(bk, slot):
            return [pltpu.make_async_copy(
                e_hbm.at[i, wid].at[pl.ds(bk * IBLK, IBLK)],
                idx_v.at[slot, i], isems[slot]) for i in (0, 1)]

        def gather(bk, k, b):
            slot = bk % 2
            return pltpu.make_async_copy(
                g_hbm.at[idx_v.at[slot, 0, k]], bufs[b], gsems[b])

        for cp in idx_copies(0, 0) + idx_copies(1, 1):
            cp.start()
        plsc.subcore_barrier()

        def do_block(bk, slot):
            for cp in idx_copies(bk, slot):
                cp.wait()
            gather(bk, 0, 0).start()
            for k in range(IBLK):
                b = k % 2
                if k + 1 < IBLK:
                    gather(bk, k + 1, (k + 1) % 2).start()
                gather(bk, k, b).wait()
                pltpu.sync_copy(bufs[b], acc.at[idx_v.at[slot, 1, k]],
                                add=True)

            def prefetch():
                for cp in idx_copies(bk + 2, slot):
                    cp.start()

            pl.when(bk + 2 < nblk)(prefetch)

        def outer(i, carry):
            do_block(2 * i, 0)
            do_block(2 * i + 1, 1)
            return carry

        lax.fori_loop(0, nblk // 2, outer, 0)
        plsc.subcore_barrier()
        pltpu.sync_copy(acc.at[pl.ds(s * rpt, rpt)],
                        out_hbm.at[c, pl.ds(s * rpt, rpt)])

    return prop


def _make_deg(np_, nchunk):
    rpt = np_ // NS
    ndeep = 4
    mesh = plsc.VectorSubcoreMesh(core_axis_name="c", subcore_axis_name="s")

    @functools.partial(
        pl.kernel,
        out_type=jax.ShapeDtypeStruct((NC, np_, D), jnp.float32),
        mesh=mesh,
        scratch_types=[
            pltpu.VMEM_SHARED((np_, D), jnp.float32),
            pltpu.VMEM((nchunk, CHUNK), jnp.int32),
            pltpu.VMEM((CHUNK, D), jnp.float32),
        ] + [pltpu.SemaphoreType.DMA] * 4,
    )
    def deg(e_hbm, ones_hbm, zrows_hbm, out_hbm, acc, idx_v, ones_v, *sems):
        c = lax.axis_index("c")
        s = lax.axis_index("s")
        wid = c * NS + s
        pltpu.sync_copy(zrows_hbm, acc.at[pl.ds(s * rpt, rpt)])
        pltpu.sync_copy(e_hbm.at[1, wid], idx_v)
        pltpu.sync_copy(ones_hbm, ones_v)
        plsc.subcore_barrier()

        def scat(j, b):
            return pltpu.make_async_copy(ones_v, acc.at[idx_v.at[j]],
                                         sems[b])

        for b in range(ndeep):
            scat(b, b).start(add=True)

        def outer(i, carry):
            for b in range(ndeep):
                j = i * ndeep + b
                scat(j, b).wait()
                nj = j + ndeep
                pl.when(nj < nchunk)(
                    lambda nj=nj, b=b: scat(nj, b).start(add=True))
            return carry

        lax.fori_loop(0, nchunk // ndeep, outer, 0)
        plsc.subcore_barrier()
        pltpu.sync_copy(acc.at[pl.ds(s * rpt, rpt)],
                        out_hbm.at[c, pl.ds(s * rpt, rpt)])

    return deg


def _mm_scale_body(x_ref, w_ref, d_ref, o_ref):
    o_ref[...] = d_ref[...] * jnp.dot(x_ref[...], w_ref[...],
                                      preferred_element_type=jnp.float32,
                                      precision=lax.Precision.HIGHEST)


def _mid_body(p_ref, g_ref, d_ref, b_ref, w_ref, o_ref):
    t = d_ref[...] * (p_ref[0] + p_ref[1] + g_ref[...]) + b_ref[...]
    t = jnp.maximum(t, 0.0)
    o_ref[...] = d_ref[...] * jnp.dot(t, w_ref[...],
                                      preferred_element_type=jnp.float32,
                                      precision=lax.Precision.HIGHEST)


def _fin_body(p_ref, g_ref, d_ref, b_ref, o_ref):
    o_ref[...] = (d_ref[...] * (p_ref[0] + p_ref[1] + g_ref[...])
                  + b_ref[...])


def _row_spec(r):
    return pl.BlockSpec((r, D), lambda i: (i, 0))


def _col_spec(r):
    return pl.BlockSpec((r, 1), lambda i: (i, 0))


def _part_spec(r):
    return pl.BlockSpec((NC, r, D), lambda i: (0, i, 0))


def _full_spec(shape):
    return pl.BlockSpec(shape, lambda i: tuple(0 for _ in shape))


def kernel(x, edge_index, W1, b1, W2, b2):
    n, d_in = x.shape
    assert d_in == D
    e = edge_index.shape[1]

    np_ = ((n + NS * 8 - 1) // (NS * 8)) * (NS * 8)
    if np_ == n:
        np_ += NS * 8
    estep = NW * CHUNK * IBLK * 2
    e_pad = ((e + estep - 1) // estep) * estep
    nchunk = e_pad // (NW * CHUNK)
    rb = n // 5 if n % 5 == 0 and (n // 5) % 8 == 0 else None
    if rb is None:
        rb = 8
        while n % (rb * 2) == 0 and rb < 2048:
            rb *= 2

    pad = n + (jnp.arange(e_pad - e, dtype=jnp.int32) % (np_ - n))
    e2 = jnp.concatenate(
        [edge_index.astype(jnp.int32), jnp.broadcast_to(pad, (2, e_pad - e))],
        axis=1).reshape(2, NW, nchunk, CHUNK)

    rpt = np_ // NS
    zrows = jnp.zeros((rpt, D), jnp.float32)
    ones = jnp.ones((CHUNK, D), jnp.float32)

    deg_k = _make_deg(np_, nchunk)
    prop_k = _make_prop(np_, nchunk)

    degp = deg_k(e2, ones, zrows)
    dcol = lax.rsqrt(degp[0, :, 0] + degp[1, :, 0] + 1.0)[:, None]

    grid = (n // rb,)
    g1 = pl.pallas_call(
        _mm_scale_body,
        grid=grid,
        in_specs=[_row_spec(rb), _full_spec((D, D)), _col_spec(rb)],
        out_specs=_row_spec(rb),
        out_shape=jax.ShapeDtypeStruct((np_, D), jnp.float32),
    )(x, W1, dcol)

    parts1 = prop_k(g1, e2, zrows)

    g2 = pl.pallas_call(
        _mid_body,
        grid=grid,
        in_specs=[_part_spec(rb), _row_spec(rb), _col_spec(rb),
                  _full_spec((1, D)), _full_spec((D, D))],
        out_specs=_row_spec(rb),
        out_shape=jax.ShapeDtypeStruct((np_, D), jnp.float32),
    )(parts1, g1, dcol, b1.reshape(1, D), W2)

    parts2 = prop_k(g2, e2, zrows)

    out = pl.pallas_call(
        _fin_body,
        grid=grid,
        in_specs=[_part_spec(rb), _row_spec(rb), _col_spec(rb),
                  _full_spec((1, D))],
        out_specs=_row_spec(rb),
        out_shape=jax.ShapeDtypeStruct((n, D), jnp.float32),
    )(parts2, g2, dcol, b2.reshape(1, D))

    return out
